# Initial kernel scaffold; baseline (speedup 1.0000x reference)
import functools
import jax
import jax.numpy as jnp
from jax import lax
from jax.experimental import pallas as pl
from jax.experimental.pallas import tpu as pltpu

K = 32
TM = 128


def _sae_body(x_ref, we_ref, wd_ref, lat_ref, rec_ref):
    x = x_ref[...]
    we = we_ref[...]
    latent = lax.dot_general(
        x, we, (((1,), (1,)), ((), ())),
        preferred_element_type=jnp.float32,
    )
    latent = jnp.maximum(latent, 0.0)

    neg = jnp.float32(-jnp.inf)

    def step(_, carry):
        work, _ = carry
        m = jnp.max(work, axis=1, keepdims=True)
        work = jnp.where(work >= m, neg, work)
        return work, m

    _, thresh = lax.fori_loop(
        0, K, step, (latent, jnp.zeros((TM, 1), jnp.float32))
    )

    sparse = jnp.where(latent >= thresh, latent, 0.0)
    lat_ref[...] = sparse

    wd = wd_ref[...]
    rec_ref[...] = lax.dot_general(
        sparse, wd, (((1,), (1,)), ((), ())),
        preferred_element_type=jnp.float32,
    )


def kernel(x, W_enc, W_dec):
    N, D = x.shape
    H = W_enc.shape[0]
    grid = (N // TM,)
    out = pl.pallas_call(
        _sae_body,
        grid=grid,
        in_specs=[
            pl.BlockSpec((TM, D), lambda i: (i, 0)),
            pl.BlockSpec((H, D), lambda i: (0, 0)),
            pl.BlockSpec((D, H), lambda i: (0, 0)),
        ],
        out_specs=[
            pl.BlockSpec((TM, H), lambda i: (i, 0)),
            pl.BlockSpec((TM, D), lambda i: (i, 0)),
        ],
        out_shape=[
            jax.ShapeDtypeStruct((N, H), jnp.float32),
            jax.ShapeDtypeStruct((N, D), jnp.float32),
        ],
        compiler_params=pltpu.CompilerParams(
            dimension_semantics=("arbitrary",),
        ),
    )(x, W_enc, W_dec)
    return (out[0], out[1])


# speedup vs baseline: 3.6667x; 3.6667x over previous
import functools
import jax
import jax.numpy as jnp
from jax import lax
from jax.experimental import pallas as pl
from jax.experimental.pallas import tpu as pltpu

K = 32
TM = 64
TM2 = 128


def _enc_body(x_ref, we_ref, lat_ref):
    x = x_ref[...]
    we = we_ref[...]
    latent = lax.dot_general(
        x, we, (((1,), (1,)), ((), ())),
        preferred_element_type=jnp.float32,
    )
    latent = jnp.maximum(latent, 0.0)

    neg = jnp.float32(-jnp.inf)

    def step(_, carry):
        work, _ = carry
        m = jnp.max(work, axis=1, keepdims=True)
        work = jnp.where(work >= m, neg, work)
        return work, m

    _, thresh = lax.fori_loop(
        0, K, step, (latent, jnp.zeros((TM, 1), jnp.float32))
    )

    lat_ref[...] = jnp.where(latent >= thresh, latent, 0.0)


def _dec_body(lat_ref, wd_ref, rec_ref):
    wd = wd_ref[...]
    rec_ref[...] = lax.dot_general(
        lat_ref[...], wd, (((1,), (1,)), ((), ())),
        preferred_element_type=jnp.float32,
    )


def kernel(x, W_enc, W_dec):
    N, D = x.shape
    H = W_enc.shape[0]

    latent_sparse = pl.pallas_call(
        _enc_body,
        grid=(N // TM,),
        in_specs=[
            pl.BlockSpec((TM, D), lambda i: (i, 0)),
            pl.BlockSpec((H, D), lambda i: (0, 0)),
        ],
        out_specs=pl.BlockSpec((TM, H), lambda i: (i, 0)),
        out_shape=jax.ShapeDtypeStruct((N, H), jnp.float32),
        compiler_params=pltpu.CompilerParams(
            dimension_semantics=("arbitrary",),
        ),
    )(x, W_enc)

    recon = pl.pallas_call(
        _dec_body,
        grid=(N // TM2,),
        in_specs=[
            pl.BlockSpec((TM2, H), lambda i: (i, 0)),
            pl.BlockSpec((D, H), lambda i: (0, 0)),
        ],
        out_specs=pl.BlockSpec((TM2, D), lambda i: (i, 0)),
        out_shape=jax.ShapeDtypeStruct((N, D), jnp.float32),
        compiler_params=pltpu.CompilerParams(
            dimension_semantics=("arbitrary",),
        ),
    )(latent_sparse, W_dec)

    return (latent_sparse, recon)
